# Initial kernel scaffold; baseline (speedup 1.0000x reference)
import functools

import jax
import jax.numpy as jnp
from jax import lax
from jax.experimental import pallas as pl
from jax.experimental.pallas import tpu as pltpu
from jax.experimental.pallas import tpu_sc as plsc

_N = 50000
_E = 800000
_D = 128
_H = 64
_HH = 32
_NPAD = 50048
_RPT = _NPAD // 16
_EPAD = 802816
_EPT = _EPAD // 16
_EK = 1024
_NCHUNK = _EPT // _EK
_J = _EK // 128
_EPT32 = _EPAD // 32
_CK = 512
_CCH = _EPT32 // _CK

_mesh = plsc.VectorSubcoreMesh(core_axis_name="c", subcore_axis_name="s")


@functools.partial(
    pl.kernel,
    out_type=jax.ShapeDtypeStruct((_NPAD, _H), jnp.float32),
    mesh=_mesh,
    scratch_types=[
        pltpu.VMEM_SHARED((_NPAD, _HH), jnp.float32),
        pltpu.VMEM((_J, 128), jnp.int32),
        pltpu.VMEM((_J, 128), jnp.int32),
        pltpu.VMEM((_EK, _HH), jnp.float32),
        pltpu.SemaphoreType.DMA,
    ],
)
def _segsum(zA, zB, src2, dst2, zrows, out, acc, srcb, dstb, rows, sem):
    c = lax.axis_index("c")
    s = lax.axis_index("s")
    pltpu.sync_copy(zrows, acc.at[pl.ds(s * _RPT, _RPT)])
    plsc.subcore_barrier()

    @pl.loop(0, _NCHUNK)
    def _chunk(k):
        rowbase = s * (_EPT // 128) + k * _J
        pltpu.sync_copy(src2.at[pl.ds(rowbase, _J)], srcb)
        pltpu.sync_copy(dst2.at[pl.ds(rowbase, _J)], dstb)

        @pl.when(c == 0)
        def _():
            cps = [
                pltpu.async_copy(zA.at[srcb.at[j]],
                                 rows.at[pl.ds(j * 128, 128)], sem)
                for j in range(_J)
            ]
            for cp in cps:
                cp.wait()

        @pl.when(c == 1)
        def _():
            cps = [
                pltpu.async_copy(zB.at[srcb.at[j]],
                                 rows.at[pl.ds(j * 128, 128)], sem)
                for j in range(_J)
            ]
            for cp in cps:
                cp.wait()

        for j in range(_J):
            pltpu.sync_copy(rows.at[pl.ds(j * 128, 128)],
                            acc.at[dstb.at[j]], add=True)

    plsc.subcore_barrier()
    pltpu.sync_copy(acc.at[pl.ds(s * _RPT, _RPT)],
                    out.at[pl.ds(s * _RPT, _RPT), pl.ds(c * _HH, _HH)])


@functools.partial(
    pl.kernel,
    out_type=jax.ShapeDtypeStruct((32, _NPAD), jnp.float32),
    mesh=_mesh,
    scratch_types=[
        pltpu.VMEM((1, _NPAD), jnp.float32),
        pltpu.VMEM((_CK,), jnp.int32),
    ],
)
def _segcount(dst1, zcnt, out, cnt, dstb):
    c = lax.axis_index("c")
    s = lax.axis_index("s")
    w = c * 16 + s
    pltpu.sync_copy(zcnt, cnt)
    ones = jnp.full((16,), 1.0, jnp.float32)
    zz = jnp.zeros((16,), jnp.int32)

    @pl.loop(0, _CCH)
    def _chunk(k):
        base = w * _EPT32 + k * _CK
        pltpu.sync_copy(dst1.at[pl.ds(base, _CK)], dstb)
        for t in range(_CK // 16):
            d = dstb[pl.ds(t * 16, 16)]
            plsc.addupdate_scatter(cnt, [zz, d], ones)

    pltpu.sync_copy(cnt, out.at[pl.ds(w, 1)])


def _proj_body(x, W, b, o):
    o[...] = jnp.dot(x[...], W[...],
                     preferred_element_type=jnp.float32) + b[...]


def _proj(x, W, b):
    n, d = x.shape
    blk = 2000
    return pl.pallas_call(
        _proj_body,
        grid=(n // blk,),
        in_specs=[
            pl.BlockSpec((blk, d), lambda i: (i, 0)),
            pl.BlockSpec((d, _H), lambda i: (0, 0)),
            pl.BlockSpec((1, _H), lambda i: (0, 0)),
        ],
        out_specs=pl.BlockSpec((blk, _H), lambda i: (i, 0)),
        out_shape=jax.ShapeDtypeStruct((n, _H), jnp.float32),
    )(x, W, b.reshape(1, _H))


def _zsplit_body(h, Wl, oA, oB):
    z = jnp.dot(h[...], Wl[...], preferred_element_type=jnp.float32)
    oA[...] = z[:, :_HH]
    oB[...] = z[:, _HH:]


def _zsplit(h, Wl):
    n = h.shape[0]
    blk = 2000
    return pl.pallas_call(
        _zsplit_body,
        grid=(n // blk,),
        in_specs=[
            pl.BlockSpec((blk, _H), lambda i: (i, 0)),
            pl.BlockSpec((_H, _H), lambda i: (0, 0)),
        ],
        out_specs=[
            pl.BlockSpec((blk, _HH), lambda i: (i, 0)),
            pl.BlockSpec((blk, _HH), lambda i: (i, 0)),
        ],
        out_shape=[
            jax.ShapeDtypeStruct((n, _HH), jnp.float32),
            jax.ShapeDtypeStruct((n, _HH), jnp.float32),
        ],
    )(h, Wl)


def _recip_body(cnt, o):
    total = jnp.sum(cnt[...], axis=0)
    o[...] = (1.0 / jnp.maximum(total, 1.0))[:, None]


def _recip(cnt32):
    blk = 2944
    return pl.pallas_call(
        _recip_body,
        grid=(_NPAD // blk,),
        in_specs=[pl.BlockSpec((32, blk), lambda i: (0, i))],
        out_specs=pl.BlockSpec((blk, 1), lambda i: (i, 0)),
        out_shape=jax.ShapeDtypeStruct((_NPAD, 1), jnp.float32),
    )(cnt32)


def _combine_body(S, r, h, Wr, b, o):
    o[...] = (S[...] * r[...]
              + jnp.dot(h[...], Wr[...], preferred_element_type=jnp.float32)
              + b[...])


def _combine_relu_body(S, r, h, Wr, b, o):
    o[...] = jnp.maximum(
        S[...] * r[...]
        + jnp.dot(h[...], Wr[...], preferred_element_type=jnp.float32)
        + b[...], 0.0)


def _combine(S, r, h, Wr, b, relu):
    n = h.shape[0]
    blk = 2000
    return pl.pallas_call(
        _combine_relu_body if relu else _combine_body,
        grid=(n // blk,),
        in_specs=[
            pl.BlockSpec((blk, _H), lambda i: (i, 0)),
            pl.BlockSpec((blk, 1), lambda i: (i, 0)),
            pl.BlockSpec((blk, _H), lambda i: (i, 0)),
            pl.BlockSpec((_H, _H), lambda i: (0, 0)),
            pl.BlockSpec((1, _H), lambda i: (0, 0)),
        ],
        out_specs=pl.BlockSpec((blk, _H), lambda i: (i, 0)),
        out_shape=jax.ShapeDtypeStruct((n, _H), jnp.float32),
    )(S, r, h, Wr, b.reshape(1, _H))


def _prep_edges(ei):
    src = ei[0].astype(jnp.int32)
    dst = ei[1].astype(jnp.int32)
    pad = _EPAD - _E
    src_p = jnp.concatenate([src, jnp.zeros((pad,), jnp.int32)])
    dst_p = jnp.concatenate([dst, jnp.full((pad,), _NPAD - 1, jnp.int32)])
    return src_p.reshape(-1, 128), dst_p.reshape(-1, 128), dst_p


def kernel(x_user, x_item, edge_index_ui, edge_index_iu,
           W_in_user, b_in_user, W_in_item, b_in_item,
           Wl_ui, bl_ui, Wr_ui, br_ui,
           Wl_iu, bl_iu, Wr_iu, br_iu):
    src2_ui, dst2_ui, dst1_ui = _prep_edges(edge_index_ui)
    src2_iu, dst2_iu, dst1_iu = _prep_edges(edge_index_iu)
    zrows = jnp.zeros((_RPT, _HH), jnp.float32)
    zcnt = jnp.zeros((1, _NPAD), jnp.float32)

    cnt_ui = _segcount(dst1_ui, zcnt)
    cnt_iu = _segcount(dst1_iu, zcnt)
    recip_i = _recip(cnt_ui)[:_N]
    recip_u = _recip(cnt_iu)[:_N]

    h_u = _proj(x_user, W_in_user, b_in_user)
    h_i = _proj(x_item, W_in_item, b_in_item)

    for l in range(2):
        zA_u, zB_u = _zsplit(h_u, Wl_ui[l])
        S_i = _segsum(zA_u, zB_u, src2_ui, dst2_ui, zrows)[:_N]
        zA_i, zB_i = _zsplit(h_i, Wl_iu[l])
        S_u = _segsum(zA_i, zB_i, src2_iu, dst2_iu, zrows)[:_N]
        relu = l < 1
        h_i_new = _combine(S_i, recip_i, h_i, Wr_ui[l],
                           bl_ui[l] + br_ui[l], relu)
        h_u_new = _combine(S_u, recip_u, h_u, Wr_iu[l],
                           bl_iu[l] + br_iu[l], relu)
        h_u, h_i = h_u_new, h_i_new
    return (h_u, h_i)


# speedup vs baseline: 7.6322x; 7.6322x over previous
import functools

import jax
import jax.numpy as jnp
from jax import lax
from jax.experimental import pallas as pl
from jax.experimental.pallas import tpu as pltpu
from jax.experimental.pallas import tpu_sc as plsc

_N = 50000
_E = 800000
_D = 128
_H = 64
_HH = 32
_NPAD = 50048
_RPT = _NPAD // 16
_EPAD = 802816
_EPT = _EPAD // 16
_EK = 512
_NCHUNK = _EPT // _EK
_J = _EK // 128
_EPT32 = _EPAD // 32
_CK = 512
_CCH = _EPT32 // _CK

_mesh = plsc.VectorSubcoreMesh(core_axis_name="c", subcore_axis_name="s")


@functools.partial(
    pl.kernel,
    out_type=jax.ShapeDtypeStruct((2 * _NPAD, _HH), jnp.float32),
    mesh=_mesh,
    scratch_types=[
        pltpu.VMEM_SHARED((_NPAD, _HH), jnp.float32),
        pltpu.VMEM((_J, 128), jnp.int32),
        pltpu.VMEM((_J, 128), jnp.int32),
        pltpu.VMEM((_EK, _HH), jnp.float32),
        pltpu.SemaphoreType.DMA,
    ],
    compiler_params=pltpu.CompilerParams(use_tc_tiling_on_sc=False),
)
def _segsum(zA, zB, src2, dst2, zrows, out, acc, srcb, dstb, rows, sem):
    c = lax.axis_index("c")
    s = lax.axis_index("s")
    pltpu.sync_copy(zrows, acc.at[pl.ds(s * _RPT, _RPT)])
    plsc.subcore_barrier()

    @pl.loop(0, _NCHUNK)
    def _chunk(k):
        rowbase = s * (_EPT // 128) + k * _J
        pltpu.sync_copy(src2.at[pl.ds(rowbase, _J)], srcb)
        pltpu.sync_copy(dst2.at[pl.ds(rowbase, _J)], dstb)

        @pl.when(c == 0)
        def _():
            cps = [
                pltpu.async_copy(zA.at[srcb.at[j]],
                                 rows.at[pl.ds(j * 128, 128)], sem)
                for j in range(_J)
            ]
            for cp in cps:
                cp.wait()

        @pl.when(c == 1)
        def _():
            cps = [
                pltpu.async_copy(zB.at[srcb.at[j]],
                                 rows.at[pl.ds(j * 128, 128)], sem)
                for j in range(_J)
            ]
            for cp in cps:
                cp.wait()

        for j in range(_J):
            pltpu.sync_copy(rows.at[pl.ds(j * 128, 128)],
                            acc.at[dstb.at[j]], add=True)

    plsc.subcore_barrier()
    pltpu.sync_copy(acc.at[pl.ds(s * _RPT, _RPT)],
                    out.at[pl.ds(c * _NPAD + s * _RPT, _RPT)])


@functools.partial(
    pl.kernel,
    out_type=jax.ShapeDtypeStruct((32 * _NPAD,), jnp.float32),
    mesh=_mesh,
    scratch_types=[
        pltpu.VMEM((_NPAD,), jnp.float32),
        pltpu.VMEM((_CK,), jnp.int32),
    ],
    compiler_params=pltpu.CompilerParams(needs_layout_passes=False),
)
def _segcount(dst1, zcnt, out, cnt, dstb):
    c = lax.axis_index("c")
    s = lax.axis_index("s")
    w = c * 16 + s
    pltpu.sync_copy(zcnt, cnt)
    ones = jnp.full((16,), 1.0, jnp.float32)

    @pl.loop(0, _CCH)
    def _chunk(k):
        base = w * _EPT32 + k * _CK
        pltpu.sync_copy(dst1.at[pl.ds(base, _CK)], dstb)
        for t in range(_CK // 16):
            d = dstb[pl.ds(t * 16, 16)]
            plsc.addupdate_scatter(cnt, [d], ones)

    pltpu.sync_copy(cnt, out.at[pl.ds(w * _NPAD, _NPAD)])


def _proj_body(x, W, b, o):
    o[...] = jnp.dot(x[...], W[...],
                     preferred_element_type=jnp.float32) + b[...]


def _proj(x, W, b):
    n, d = x.shape
    blk = 2000
    return pl.pallas_call(
        _proj_body,
        grid=(n // blk,),
        in_specs=[
            pl.BlockSpec((blk, d), lambda i: (i, 0)),
            pl.BlockSpec((d, _H), lambda i: (0, 0)),
            pl.BlockSpec((1, _H), lambda i: (0, 0)),
        ],
        out_specs=pl.BlockSpec((blk, _H), lambda i: (i, 0)),
        out_shape=jax.ShapeDtypeStruct((n, _H), jnp.float32),
    )(x, W, b.reshape(1, _H))


def _zsplit_body(h, Wl, oA, oB):
    z = jnp.dot(h[...], Wl[...], preferred_element_type=jnp.float32)
    oA[...] = z[:, :_HH]
    oB[...] = z[:, _HH:]


def _zsplit(h, Wl):
    n = h.shape[0]
    blk = 2000
    return pl.pallas_call(
        _zsplit_body,
        grid=(n // blk,),
        in_specs=[
            pl.BlockSpec((blk, _H), lambda i: (i, 0)),
            pl.BlockSpec((_H, _H), lambda i: (0, 0)),
        ],
        out_specs=[
            pl.BlockSpec((blk, _HH), lambda i: (i, 0)),
            pl.BlockSpec((blk, _HH), lambda i: (i, 0)),
        ],
        out_shape=[
            jax.ShapeDtypeStruct((n, _HH), jnp.float32),
            jax.ShapeDtypeStruct((n, _HH), jnp.float32),
        ],
    )(h, Wl)


def _recip_body(cnt, o):
    total = jnp.sum(cnt[...], axis=0)
    o[...] = (1.0 / jnp.maximum(total, 1.0))[:, None]


def _recip(cnt32):
    blk = 2944
    return pl.pallas_call(
        _recip_body,
        grid=(_NPAD // blk,),
        in_specs=[pl.BlockSpec((32, blk), lambda i: (0, i))],
        out_specs=pl.BlockSpec((blk, 1), lambda i: (i, 0)),
        out_shape=jax.ShapeDtypeStruct((_NPAD, 1), jnp.float32),
    )(cnt32)


def _combine_body(S, r, h, Wr, b, o):
    o[...] = (S[...] * r[...]
              + jnp.dot(h[...], Wr[...], preferred_element_type=jnp.float32)
              + b[...])


def _combine_relu_body(S, r, h, Wr, b, o):
    o[...] = jnp.maximum(
        S[...] * r[...]
        + jnp.dot(h[...], Wr[...], preferred_element_type=jnp.float32)
        + b[...], 0.0)


def _combine(S, r, h, Wr, b, relu):
    n = h.shape[0]
    blk = 2000
    return pl.pallas_call(
        _combine_relu_body if relu else _combine_body,
        grid=(n // blk,),
        in_specs=[
            pl.BlockSpec((blk, _H), lambda i: (i, 0)),
            pl.BlockSpec((blk, 1), lambda i: (i, 0)),
            pl.BlockSpec((blk, _H), lambda i: (i, 0)),
            pl.BlockSpec((_H, _H), lambda i: (0, 0)),
            pl.BlockSpec((1, _H), lambda i: (0, 0)),
        ],
        out_specs=pl.BlockSpec((blk, _H), lambda i: (i, 0)),
        out_shape=jax.ShapeDtypeStruct((n, _H), jnp.float32),
    )(S, r, h, Wr, b.reshape(1, _H))


def _prep_edges(ei):
    src = ei[0].astype(jnp.int32)
    dst = ei[1].astype(jnp.int32)
    pad = _EPAD - _E
    src_p = jnp.concatenate([src, jnp.zeros((pad,), jnp.int32)])
    dst_p = jnp.concatenate([dst, jnp.full((pad,), _NPAD - 1, jnp.int32)])
    return src_p.reshape(-1, 128), dst_p.reshape(-1, 128), dst_p


def kernel(x_user, x_item, edge_index_ui, edge_index_iu,
           W_in_user, b_in_user, W_in_item, b_in_item,
           Wl_ui, bl_ui, Wr_ui, br_ui,
           Wl_iu, bl_iu, Wr_iu, br_iu):
    src2_ui, dst2_ui, dst1_ui = _prep_edges(edge_index_ui)
    src2_iu, dst2_iu, dst1_iu = _prep_edges(edge_index_iu)
    zrows = jnp.zeros((_RPT, _HH), jnp.float32)
    zcnt = jnp.zeros((_NPAD,), jnp.float32)

    cnt_ui = _segcount(dst1_ui, zcnt).reshape(32, _NPAD)
    cnt_iu = _segcount(dst1_iu, zcnt).reshape(32, _NPAD)
    recip_i = _recip(cnt_ui)[:_N]
    recip_u = _recip(cnt_iu)[:_N]

    h_u = _proj(x_user, W_in_user, b_in_user)
    h_i = _proj(x_item, W_in_item, b_in_item)

    for l in range(2):
        zA_u, zB_u = _zsplit(h_u, Wl_ui[l])
        Sh_i = _segsum(zA_u, zB_u, src2_ui, dst2_ui, zrows)
        S_i = jnp.concatenate([Sh_i[:_N], Sh_i[_NPAD:_NPAD + _N]], axis=1)
        zA_i, zB_i = _zsplit(h_i, Wl_iu[l])
        Sh_u = _segsum(zA_i, zB_i, src2_iu, dst2_iu, zrows)
        S_u = jnp.concatenate([Sh_u[:_N], Sh_u[_NPAD:_NPAD + _N]], axis=1)
        relu = l < 1
        h_i_new = _combine(S_i, recip_i, h_i, Wr_ui[l],
                           bl_ui[l] + br_ui[l], relu)
        h_u_new = _combine(S_u, recip_u, h_u, Wr_iu[l],
                           bl_iu[l] + br_iu[l], relu)
        h_u, h_i = h_u_new, h_i_new
    return (h_u, h_i)
